# baseline (speedup 1.0000x reference)
import functools

import jax
import jax.numpy as jnp
from jax import lax
from jax.experimental import pallas as pl
from jax.experimental.pallas import tpu as pltpu
from jax.experimental.pallas import tpu_sc as plsc

P = 4
NPROB = 200
NORD = 200
NREL = 3
E = 300
B = 16384
GROUP = 4

NC = 2
NS = 16
L = 16
NW = NC * NS
BPW = B // NW
TBL = NREL * NPROB * NORD


LW = 128


def _tables_body(prob_ref, rel_ref, ord_ref, w_ref, pidx_ref, ridx_ref,
                 tidx_ref, out_ref, fidx_ref):
    for k in range(NREL):
        acc = jnp.zeros((NPROB, NORD), jnp.float32)
        for p in range(P):
            lhs = prob_ref[p] * rel_ref[k, p : p + 1, :] * w_ref[0:1, p : p + 1]
            acc = acc + lax.dot_general(
                lhs,
                ord_ref[p],
                (((1,), (1,)), ((), ())),
                preferred_element_type=jnp.float32,
            )
        out_ref[pl.ds((2 * k) * NPROB, NPROB), :] = acc[:, 0:LW]
        out_ref[pl.ds((2 * k + 1) * NPROB, NPROB), 0 : NORD - LW] = acc[:, LW:NORD]
    t = tidx_ref[...]
    fidx_ref[...] = (
        ridx_ref[...] * (2 * NPROB * LW)
        + (t >> 7) * (NPROB * LW)
        + pidx_ref[...] * LW
        + (t & (LW - 1))
    )


def _build_tables(prob_tables, rel_tables, ord_tables, final_w,
                  problems, rels, targets):
    return pl.pallas_call(
        _tables_body,
        out_shape=(
            jax.ShapeDtypeStruct((NREL * 2 * NPROB, LW), jnp.float32),
            jax.ShapeDtypeStruct((B // LW, LW), jnp.int32),
        ),
        in_specs=[pl.BlockSpec(memory_space=pltpu.VMEM)] * 7,
        out_specs=(
            pl.BlockSpec(memory_space=pltpu.VMEM),
            pl.BlockSpec(memory_space=pltpu.VMEM),
        ),
    )(prob_tables, rel_tables.transpose(1, 0, 2), ord_tables, final_w,
      problems.reshape(B // LW, LW), rels.reshape(B // LW, LW),
      targets.reshape(B // LW, LW))


@functools.partial(
    pl.kernel,
    out_type=jax.ShapeDtypeStruct((B,), jnp.float32),
    mesh=plsc.VectorSubcoreMesh(
        core_axis_name="c", subcore_axis_name="s", num_cores=NC, num_subcores=NS
    ),
    compiler_params=pltpu.CompilerParams(needs_layout_passes=False),
    scratch_types=[
        pltpu.VMEM((BPW,), jnp.int32),
        pltpu.VMEM((BPW,), jnp.float32),
        pltpu.SemaphoreType.DMA,
    ],
)
def _sc_gather(table_hbm, fidx_hbm, out_hbm, fidx_v, vals_v, sem):
    wid = lax.axis_index("s") * NC + lax.axis_index("c")
    base = wid * BPW
    pltpu.async_copy(fidx_hbm.at[pl.ds(base, BPW)], fidx_v, sem).wait()
    pltpu.async_copy(table_hbm.at[fidx_v], vals_v, sem).wait()
    pltpu.sync_copy(vals_v, out_hbm.at[pl.ds(base, BPW)])


def _loss_body(s_ref, out_ref):
    s = s_ref[...]
    c = lax.broadcasted_iota(jnp.int32, (LW, LW), 0)
    cp = lax.broadcasted_iota(jnp.int32, (LW, LW), 1)
    pmat = (c == (cp & ~(GROUP - 1))).astype(jnp.float32)
    pos = lax.dot_general(s, pmat, (((1,), (0,)), ((), ())),
                          preferred_element_type=jnp.float32)
    neg_mask = ((cp & (GROUP - 1)) != 0).astype(jnp.float32)[0:1, :]
    contrib = jnp.maximum(s - pos + 1.0, 0.0) * neg_mask
    out_ref[0, 0] = jnp.sum(contrib) * (GROUP / B)


def _reduce(scores2d):
    return pl.pallas_call(
        _loss_body,
        out_shape=jax.ShapeDtypeStruct((1, 1), jnp.float32),
        in_specs=[pl.BlockSpec(memory_space=pltpu.VMEM)],
        out_specs=pl.BlockSpec(memory_space=pltpu.SMEM),
    )(scores2d)


def kernel(problems, rels, targets, labels, prob_tables, ord_tables,
           rel_tables, final_w):
    del labels
    sw, fidx = _build_tables(
        prob_tables, rel_tables, ord_tables, final_w,
        problems.astype(jnp.int32), rels.astype(jnp.int32),
        targets.astype(jnp.int32),
    )
    table = sw.reshape(NREL * 2 * NPROB * LW)
    scores = _sc_gather(table, fidx.reshape(B))
    loss = _reduce(scores.reshape(B // LW, LW))
    return loss[0, 0]


# speedup vs baseline: 1.0126x; 1.0126x over previous
import functools

import jax
import jax.numpy as jnp
from jax import lax
from jax.experimental import pallas as pl
from jax.experimental.pallas import tpu as pltpu
from jax.experimental.pallas import tpu_sc as plsc

P = 4
NPROB = 200
NORD = 200
NREL = 3
E = 300
B = 16384
GROUP = 4

NC = 2
NS = 16
L = 16
NW = NC * NS
BPW = B // NW


LW = 128


def _tables_body(prob_ref, rel_ref, ord_ref, w_ref, pidx_ref, ridx_ref,
                 tidx_ref, out_ref, fidx_ref):
    for k in range(NREL):
        acc = jnp.zeros((NPROB, NORD), jnp.float32)
        for p in range(P):
            lhs = prob_ref[p] * rel_ref[k, p : p + 1, :] * w_ref[0:1, p : p + 1]
            acc = acc + lax.dot_general(
                lhs,
                ord_ref[p],
                (((1,), (1,)), ((), ())),
                preferred_element_type=jnp.float32,
            )
        out_ref[pl.ds((2 * k) * NPROB, NPROB), :] = acc[:, 0:LW]
        out_ref[pl.ds((2 * k + 1) * NPROB, NPROB), 0 : NORD - LW] = acc[:, LW:NORD]
    t = tidx_ref[...]
    fidx_ref[...] = (
        ridx_ref[...] * (2 * NPROB * LW)
        + (t >> 7) * (NPROB * LW)
        + pidx_ref[...] * LW
        + (t & (LW - 1))
    )


def _build_tables(prob_tables, rel_tables, ord_tables, final_w,
                  problems, rels, targets):
    return pl.pallas_call(
        _tables_body,
        out_shape=(
            jax.ShapeDtypeStruct((NREL * 2 * NPROB, LW), jnp.float32),
            jax.ShapeDtypeStruct((B // LW, LW), jnp.int32),
        ),
        in_specs=[pl.BlockSpec(memory_space=pltpu.VMEM)] * 7,
        out_specs=(
            pl.BlockSpec(memory_space=pltpu.VMEM),
            pl.BlockSpec(memory_space=pltpu.VMEM),
        ),
    )(prob_tables, rel_tables.transpose(1, 0, 2), ord_tables, final_w,
      problems.reshape(B // LW, LW), rels.reshape(B // LW, LW),
      targets.reshape(B // LW, LW))


@functools.partial(
    pl.kernel,
    out_type=jax.ShapeDtypeStruct((NW * L,), jnp.float32),
    mesh=plsc.VectorSubcoreMesh(
        core_axis_name="c", subcore_axis_name="s", num_cores=NC, num_subcores=NS
    ),
    compiler_params=pltpu.CompilerParams(needs_layout_passes=False),
    scratch_types=[
        pltpu.VMEM((BPW,), jnp.int32),
        pltpu.VMEM((BPW,), jnp.float32),
        pltpu.VMEM((L,), jnp.float32),
        pltpu.SemaphoreType.DMA,
    ],
)
def _sc_gather_loss(table_hbm, fidx_hbm, out_hbm, fidx_v, vals_v, acc_v, sem):
    wid = lax.axis_index("s") * NC + lax.axis_index("c")
    base = wid * BPW
    pltpu.async_copy(fidx_hbm.at[pl.ds(base, BPW)], fidx_v, sem).wait()
    pltpu.async_copy(table_hbm.at[fidx_v], vals_v, sem).wait()

    def _loss(j, acc):
        i0 = (lax.iota(jnp.int32, L) + j * L) * GROUP
        pos = plsc.load_gather(vals_v, [i0])
        n1 = plsc.load_gather(vals_v, [i0 + 1])
        n2 = plsc.load_gather(vals_v, [i0 + 2])
        n3 = plsc.load_gather(vals_v, [i0 + 3])
        return (acc
                + jnp.maximum(n1 - pos + 1.0, 0.0)
                + jnp.maximum(n2 - pos + 1.0, 0.0)
                + jnp.maximum(n3 - pos + 1.0, 0.0))

    acc = lax.fori_loop(0, BPW // GROUP // L, _loss,
                        jnp.zeros((L,), jnp.float32), unroll=False)
    acc_v[...] = acc
    pltpu.sync_copy(acc_v, out_hbm.at[pl.ds(wid * L, L)])


def _reduce_body(part_ref, out_ref):
    out_ref[0, 0] = jnp.sum(part_ref[...]) * (GROUP / B)


def _reduce(partials):
    return pl.pallas_call(
        _reduce_body,
        out_shape=jax.ShapeDtypeStruct((1, 1), jnp.float32),
        in_specs=[pl.BlockSpec(memory_space=pltpu.VMEM)],
        out_specs=pl.BlockSpec(memory_space=pltpu.SMEM),
    )(partials)


def kernel(problems, rels, targets, labels, prob_tables, ord_tables,
           rel_tables, final_w):
    del labels
    sw, fidx = _build_tables(
        prob_tables, rel_tables, ord_tables, final_w,
        problems.astype(jnp.int32), rels.astype(jnp.int32),
        targets.astype(jnp.int32),
    )
    table = sw.reshape(NREL * 2 * NPROB * LW)
    partials = _sc_gather_loss(table, fidx.reshape(B))
    loss = _reduce(partials)
    return loss[0, 0]
